# Initial kernel scaffold; baseline (speedup 1.0000x reference)
import jax
import jax.numpy as jnp
from jax.experimental import pallas as pl


def kernel(x, row_embed):
    raise NotImplementedError("write your pallas kernel here")


# speedup vs baseline: 1.9744x; 1.9744x over previous
import jax
import jax.numpy as jnp
from jax.experimental import pallas as pl
from jax.experimental.pallas import tpu as pltpu

_LB = 512


def _add_kernel(x_ref, row_ref, o_ref):
    o_ref[0, :, :] = x_ref[0, :, :] + row_ref[:, :]


def kernel(x, row_embed):
    B, L, D = x.shape
    table = row_embed[:L]
    grid = (L // _LB, B)
    return pl.pallas_call(
        _add_kernel,
        grid=grid,
        in_specs=[
            pl.BlockSpec((1, _LB, D), lambda l, b: (b, l, 0)),
            pl.BlockSpec((_LB, D), lambda l, b: (l, 0)),
        ],
        out_specs=pl.BlockSpec((1, _LB, D), lambda l, b: (b, l, 0)),
        out_shape=jax.ShapeDtypeStruct((B, L, D), x.dtype),
        compiler_params=pltpu.CompilerParams(
            dimension_semantics=("parallel", "arbitrary"),
        ),
    )(x, table)


# speedup vs baseline: 2.2096x; 1.1191x over previous
import jax
import jax.numpy as jnp
from jax.experimental import pallas as pl
from jax.experimental.pallas import tpu as pltpu

_LB = 1024


def _add_kernel(x_ref, row_ref, o_ref):
    o_ref[0, :, :] = x_ref[0, :, :] + row_ref[:, :]


def kernel(x, row_embed):
    B, L, D = x.shape
    table = row_embed[:L]
    grid = (L // _LB, B)
    return pl.pallas_call(
        _add_kernel,
        grid=grid,
        in_specs=[
            pl.BlockSpec((1, _LB, D), lambda l, b: (b, l, 0)),
            pl.BlockSpec((_LB, D), lambda l, b: (l, 0)),
        ],
        out_specs=pl.BlockSpec((1, _LB, D), lambda l, b: (b, l, 0)),
        out_shape=jax.ShapeDtypeStruct((B, L, D), x.dtype),
        compiler_params=pltpu.CompilerParams(
            dimension_semantics=("parallel", "arbitrary"),
        ),
    )(x, table)


# speedup vs baseline: 2.4372x; 1.1030x over previous
import jax
import jax.numpy as jnp
from jax.experimental import pallas as pl
from jax.experimental.pallas import tpu as pltpu

_LB = 2048


def _add_kernel(x_ref, row_ref, o_ref):
    o_ref[0, :, :] = x_ref[0, :, :] + row_ref[:, :]


def kernel(x, row_embed):
    B, L, D = x.shape
    table = row_embed[:L]
    grid = (L // _LB, B)
    return pl.pallas_call(
        _add_kernel,
        grid=grid,
        in_specs=[
            pl.BlockSpec((1, _LB, D), lambda l, b: (b, l, 0)),
            pl.BlockSpec((_LB, D), lambda l, b: (l, 0)),
        ],
        out_specs=pl.BlockSpec((1, _LB, D), lambda l, b: (b, l, 0)),
        out_shape=jax.ShapeDtypeStruct((B, L, D), x.dtype),
        compiler_params=pltpu.CompilerParams(
            dimension_semantics=("parallel", "arbitrary"),
        ),
    )(x, table)
